# Initial kernel scaffold; baseline (speedup 1.0000x reference)
import jax
import jax.numpy as jnp
from jax.experimental import pallas as pl


def kernel(pred_poses):
    raise NotImplementedError("write your pallas kernel here")


# speedup vs baseline: 11.5752x; 11.5752x over previous
import jax
import jax.numpy as jnp
from jax.experimental import pallas as pl
from jax.experimental.pallas import tpu as pltpu


def _diversity_kernel(x_ref, mean_ref, std_ref, cv_ref, nnd_ref):
    B, N, D = x_ref.shape

    def body(b, _):
        x = x_ref[b]
        g = jnp.dot(x, x.T, preferred_element_type=jnp.float32)
        row = jax.lax.broadcasted_iota(jnp.int32, (N, N), 0)
        col = jax.lax.broadcasted_iota(jnp.int32, (N, N), 1)
        eye = row == col
        diag = jnp.where(eye, g, 0.0)
        sq_i = jnp.sum(diag, axis=1, keepdims=True)
        sq_j = jnp.sum(diag, axis=0, keepdims=True)
        d2 = sq_i + sq_j - 2.0 * g
        d2 = jnp.where(eye, jnp.inf, d2)
        mind2 = jnp.min(d2, axis=0, keepdims=True)
        nnd_ref[pl.ds(b, 1), :] = jnp.sqrt(jnp.maximum(mind2, 0.0))
        return 0

    jax.lax.fori_loop(0, B, body, 0, unroll=True)

    nnd = nnd_ref[...]
    m = B * N
    mean = jnp.sum(nnd) / m
    var = jnp.sum((nnd - mean) ** 2) / (m - 1)
    std = jnp.sqrt(var)
    cv = jnp.where(mean > 1e-08, std / jnp.maximum(mean, 1e-08), 0.0)
    mean_ref[0, 0] = mean
    std_ref[0, 0] = std
    cv_ref[0, 0] = cv


def kernel(pred_poses):
    B, N, D = pred_poses.shape
    scalar = jax.ShapeDtypeStruct((1, 1), jnp.float32)
    mean, std, cv = pl.pallas_call(
        _diversity_kernel,
        out_shape=(scalar, scalar, scalar),
        in_specs=[pl.BlockSpec(memory_space=pltpu.VMEM)],
        out_specs=(
            pl.BlockSpec(memory_space=pltpu.SMEM),
            pl.BlockSpec(memory_space=pltpu.SMEM),
            pl.BlockSpec(memory_space=pltpu.SMEM),
        ),
        scratch_shapes=[pltpu.VMEM((B, N), jnp.float32)],
    )(pred_poses)
    return (mean[0, 0], std[0, 0], cv[0, 0])
